# baseline (speedup 1.0000x reference)
import functools

import jax
import jax.numpy as jnp
from jax import lax
from jax.experimental import pallas as pl
from jax.experimental.pallas import tpu as pltpu
from jax.experimental.pallas import tpu_sc as plsc


def _make_gather(B, V, Da, Db):
    info = plsc.get_sparse_core_info()
    NW = info.num_cores * info.num_subcores
    assert B % (8 * NW) == 0
    b_per_w = B // NW
    mesh = plsc.VectorSubcoreMesh(core_axis_name="c", subcore_axis_name="s")

    @functools.partial(
        pl.kernel,
        mesh=mesh,
        compiler_params=pltpu.CompilerParams(use_tc_tiling_on_sc=False),
        out_type=(
            jax.ShapeDtypeStruct((B, Da), jnp.float32),
            jax.ShapeDtypeStruct((B, Db), jnp.float32),
        ),
        scratch_types=[
            pltpu.VMEM((b_per_w,), jnp.int32),
            pltpu.VMEM((b_per_w, Da), jnp.float32),
            pltpu.VMEM((b_per_w, Db), jnp.float32),
            pltpu.SemaphoreType.DMA,
            pltpu.SemaphoreType.DMA,
        ],
    )
    def gather_k(idx_hbm, a_hbm, b_hbm, out_a_hbm, out_b_hbm,
                 idx_v, rows_a, rows_b, sem_a, sem_b):
        wid = lax.axis_index("s") * info.num_cores + lax.axis_index("c")
        base = wid * b_per_w
        pltpu.sync_copy(idx_hbm.at[pl.ds(base, b_per_w)], idx_v)
        cp_a = pltpu.async_copy(a_hbm.at[idx_v], rows_a, sem_a)
        cp_b = pltpu.async_copy(b_hbm.at[idx_v], rows_b, sem_b)
        cp_a.wait()
        pltpu.sync_copy(rows_a, out_a_hbm.at[pl.ds(base, b_per_w)])
        cp_b.wait()
        pltpu.sync_copy(rows_b, out_b_hbm.at[pl.ds(base, b_per_w)])

    return gather_k


def kernel(indices, values_a, values_b):
    B = indices.shape[0]
    V, Da = values_a.shape
    Db = values_b.shape[1]
    gather_k = _make_gather(B, V, Da, Db)
    out_a, out_b = gather_k(indices[:, 0], values_a, values_b)
    return (out_a, out_b)
